# baseline (speedup 1.0000x reference)
import functools

import jax
import jax.numpy as jnp
from jax import lax
from jax.experimental import pallas as pl
from jax.experimental.pallas import tpu as pltpu
from jax.experimental.pallas import tpu_sc as plsc

N = 10000
E = 320000
F = 128
H = 256

NC = 2
NS = 16
NW = NC * NS
CHUNK = 128
JB = 40

OUTER1 = 2
EPAD1 = NW * OUTER1 * JB * CHUNK

OUTER2 = 4
EPAD2 = NS * OUTER2 * JB * CHUNK

ACC_ROWS = N
NZ = N + 8
ROWS_PER_TILE = 624

_MESH = plsc.VectorSubcoreMesh(core_axis_name="c", subcore_axis_name="s")


def _gather_scatter_loop(table_hbm, accum, lead, src_hbm, dst_hbm, src_all,
                         dst_all, rows0, rows1, sem0, sem1, outer):
    npair = JB // 2

    def gather(i, buf, sem):
        return pltpu.async_copy(table_hbm.at[src_all.at[i]], buf, sem)

    def wait(i, buf, sem):
        pltpu.make_async_copy(table_hbm.at[src_all.at[i]], buf, sem).wait()

    def scatter(i, buf):
        pltpu.sync_copy(buf, accum.at[dst_all.at[i]], add=True)

    def outer_body(ob, carry):
        pltpu.sync_copy(src_hbm.at[lead, ob], src_all)
        pltpu.sync_copy(dst_hbm.at[lead, ob], dst_all)
        gather(0, rows0, sem0)

        def step(j, c2):
            i0 = j * 2
            gather(i0 + 1, rows1, sem1)
            wait(i0, rows0, sem0)
            scatter(i0, rows0)

            @pl.when(j + 1 < npair)
            def _():
                gather(i0 + 2, rows0, sem0)

            wait(i0 + 1, rows1, sem1)
            scatter(i0 + 1, rows1)
            return c2

        lax.fori_loop(0, npair, step, 0)
        return carry

    lax.fori_loop(0, outer, outer_body, 0)


def _copy_out_stripe(accum, out_slice_fn, s):
    @pl.when(s < NS - 1)
    def _():
        r0 = pl.multiple_of(s * ROWS_PER_TILE, 8)
        pltpu.sync_copy(accum.at[pl.ds(r0, ROWS_PER_TILE)],
                        out_slice_fn(r0, ROWS_PER_TILE))

    @pl.when(s == NS - 1)
    def _():
        r0 = (NS - 1) * ROWS_PER_TILE
        pltpu.sync_copy(accum.at[pl.ds(r0, N - r0)], out_slice_fn(r0, N - r0))


@functools.partial(
    pl.kernel,
    out_type=jax.ShapeDtypeStruct((NC, N, F), jnp.float32),
    mesh=_MESH,
    scratch_types=[
        pltpu.VMEM_SHARED((ACC_ROWS, F), jnp.float32),
        pltpu.VMEM((JB, CHUNK), jnp.int32),
        pltpu.VMEM((JB, CHUNK), jnp.int32),
        pltpu.VMEM((CHUNK, F), jnp.float32),
        pltpu.VMEM((CHUNK, F), jnp.float32),
        pltpu.SemaphoreType.DMA,
        pltpu.SemaphoreType.DMA,
    ],
)
def _sc_agg1(xx_hbm, srca_hbm, dsta_hbm, srcb_hbm, dstb_hbm, zeros_hbm,
             out_hbm, accum, src_all, dst_all, rows0, rows1, sem0, sem1):
    c = lax.axis_index("c")
    s = lax.axis_index("s")

    @pl.when(s == 0)
    def _():
        pltpu.sync_copy(zeros_hbm, accum)

    plsc.subcore_barrier()

    @pl.when(c == 0)
    def _():
        _gather_scatter_loop(xx_hbm.at[0], accum, s, srca_hbm, dsta_hbm,
                             src_all, dst_all, rows0, rows1, sem0, sem1,
                             OUTER1)

    @pl.when(c == 1)
    def _():
        _gather_scatter_loop(xx_hbm.at[1], accum, s, srcb_hbm, dstb_hbm,
                             src_all, dst_all, rows0, rows1, sem0, sem1,
                             OUTER1)

    plsc.subcore_barrier()
    _copy_out_stripe(accum, lambda r0, n: out_hbm.at[c, pl.ds(r0, n)], s)


@functools.partial(
    pl.kernel,
    out_type=jax.ShapeDtypeStruct((NC, N, F), jnp.float32),
    mesh=_MESH,
    scratch_types=[
        pltpu.VMEM_SHARED((ACC_ROWS, F), jnp.float32),
        pltpu.VMEM((JB, CHUNK), jnp.int32),
        pltpu.VMEM((JB, CHUNK), jnp.int32),
        pltpu.VMEM((CHUNK, F), jnp.float32),
        pltpu.VMEM((CHUNK, F), jnp.float32),
        pltpu.SemaphoreType.DMA,
        pltpu.SemaphoreType.DMA,
    ],
)
def _sc_agg2(ha_hbm, hb_hbm, src_hbm, dst_hbm, zeros_hbm, out_hbm,
             accum, src_all, dst_all, rows0, rows1, sem0, sem1):
    c = lax.axis_index("c")
    s = lax.axis_index("s")

    @pl.when(s == 0)
    def _():
        pltpu.sync_copy(zeros_hbm, accum)

    plsc.subcore_barrier()

    @pl.when(c == 0)
    def _():
        _gather_scatter_loop(ha_hbm, accum, s, src_hbm, dst_hbm, src_all,
                             dst_all, rows0, rows1, sem0, sem1, OUTER2)

    @pl.when(c == 1)
    def _():
        _gather_scatter_loop(hb_hbm, accum, s, src_hbm, dst_hbm, src_all,
                             dst_all, rows0, rows1, sem0, sem1, OUTER2)

    plsc.subcore_barrier()
    _copy_out_stripe(accum, lambda r0, n: out_hbm.at[c, pl.ds(r0, n)], s)


RB = 1000
GRID = N // RB

_row_spec = pl.BlockSpec((RB, F), lambda i: (i, 0))
_w_spec = pl.BlockSpec((F, H), lambda i: (0, 0))
_b_spec = pl.BlockSpec((1, H), lambda i: (0, 0))


def _tc1_body(a0, a1, x, w_rel, w_root, b, oa, ob):
    agg = a0[...] + a1[...]
    h = (jnp.dot(agg, w_rel[...], preferred_element_type=jnp.float32)
         + jnp.dot(x[...], w_root[...], preferred_element_type=jnp.float32)
         + b[...])
    h = jnp.maximum(h, 0.0)
    oa[...] = h[:, :F]
    ob[...] = h[:, F:]


def _tc1(a0, a1, x, w_rel, w_root, b):
    return pl.pallas_call(
        _tc1_body,
        grid=(GRID,),
        in_specs=[_row_spec, _row_spec, _row_spec, _w_spec, _w_spec, _b_spec],
        out_specs=[_row_spec, _row_spec],
        out_shape=[jax.ShapeDtypeStruct((N, F), jnp.float32)] * 2,
    )(a0, a1, x, w_rel, w_root, b)


def _tc2_body(aa, ab, ha, hb, wr0, wr1, wq0, wq1, b, o):
    o[...] = (jnp.dot(aa[...], wr0[...], preferred_element_type=jnp.float32)
              + jnp.dot(ab[...], wr1[...], preferred_element_type=jnp.float32)
              + jnp.dot(ha[...], wq0[...], preferred_element_type=jnp.float32)
              + jnp.dot(hb[...], wq1[...], preferred_element_type=jnp.float32)
              + b[...])


def _tc2(aa, ab, ha, hb, wr0, wr1, wq0, wq1, b):
    return pl.pallas_call(
        _tc2_body,
        grid=(GRID,),
        in_specs=[_row_spec] * 4 + [_w_spec] * 4 + [_b_spec],
        out_specs=pl.BlockSpec((RB, H), lambda i: (i, 0)),
        out_shape=jax.ShapeDtypeStruct((N, H), jnp.float32),
    )(aa, ab, ha, hb, wr0, wr1, wq0, wq1, b)


def _pad_edges(src, dst, epad, lead):
    pad = epad - E
    srcp = jnp.concatenate([src, jnp.full((pad,), N, jnp.int32)])
    dstp = jnp.concatenate([dst, jnp.arange(pad, dtype=jnp.int32) % N])
    return (srcp.reshape(lead, -1, JB, CHUNK), dstp.reshape(lead, -1, JB, CHUNK))


def kernel(x, edge_index, W1_rel, b1_rel, W1_root, W2_rel, b2_rel, W2_root):
    src = edge_index[0].astype(jnp.int32)
    dst = edge_index[1].astype(jnp.int32)
    src1, dst1 = _pad_edges(src, dst, EPAD1, NC * NS)
    src1 = src1.reshape(NC, NS, OUTER1, JB, CHUNK)
    dst1 = dst1.reshape(NC, NS, OUTER1, JB, CHUNK)
    src2, dst2 = _pad_edges(src, dst, EPAD2, NS)
    zeros = jnp.zeros((ACC_ROWS, F), jnp.float32)
    zrows = jnp.zeros((NZ - N, F), jnp.float32)

    b1 = b1_rel.reshape(1, H)
    b2 = b2_rel.reshape(1, H)

    xp = jnp.concatenate([x, zrows])
    xx = jnp.stack([xp, xp])
    p1 = _sc_agg1(xx, src1[0], dst1[0], src1[1], dst1[1], zeros)
    h1a, h1b = _tc1(p1[0], p1[1], x, W1_rel, W1_root, b1)

    a2 = _sc_agg2(jnp.concatenate([h1a, zrows]), jnp.concatenate([h1b, zrows]),
                  src2, dst2, zeros)

    out = _tc2(a2[0], a2[1], h1a, h1b,
               W2_rel[:F], W2_rel[F:], W2_root[:F], W2_root[F:], b2)
    return out


# speedup vs baseline: 3.1854x; 3.1854x over previous
import functools

import jax
import jax.numpy as jnp
from jax import lax
from jax.experimental import pallas as pl
from jax.experimental.pallas import tpu as pltpu
from jax.experimental.pallas import tpu_sc as plsc

N = 10000
E = 320000
F = 128
H = 256

NC = 2
NS = 16
NW = NC * NS
CHUNK = 128
JB = 40

OUTER1 = 2
EPAD1 = NW * OUTER1 * JB * CHUNK

OUTER2 = 4
EPAD2 = NS * OUTER2 * JB * CHUNK

ACC_ROWS = N
CH1 = 80
CH1_LAST = 20
CH2 = 160
CH2_LAST = 100
ROWS_PER_TILE = 624

_MESH = plsc.VectorSubcoreMesh(core_axis_name="c", subcore_axis_name="s")


def _gather_scatter_loop(table_hbm, accum, lead, src_hbm, dst_hbm, src_all,
                         dst_all, rows0, rows1, sem0, sem1, total_chunks):
    def gather(i, buf, sem):
        return pltpu.async_copy(table_hbm.at[src_all.at[i]], buf, sem)

    def wait(i, buf, sem):
        pltpu.make_async_copy(table_hbm.at[src_all.at[i]], buf, sem).wait()

    def scatter(i, buf):
        pltpu.sync_copy(buf, accum.at[dst_all.at[i]], add=True)

    total_chunks = jnp.int32(total_chunks)
    nblocks = (total_chunks + (JB - 1)) // JB

    def outer_body(ob, carry):
        pltpu.sync_copy(src_hbm.at[lead, ob], src_all)
        pltpu.sync_copy(dst_hbm.at[lead, ob], dst_all)
        npair = jnp.minimum(JB, total_chunks - ob * JB) // 2
        gather(0, rows0, sem0)

        def step(j, c2):
            i0 = j * 2
            gather(i0 + 1, rows1, sem1)
            wait(i0, rows0, sem0)
            scatter(i0, rows0)

            @pl.when(j + 1 < npair)
            def _():
                gather(i0 + 2, rows0, sem0)

            wait(i0 + 1, rows1, sem1)
            scatter(i0 + 1, rows1)
            return c2

        lax.fori_loop(0, npair, step, 0)
        return carry

    lax.fori_loop(0, nblocks, outer_body, 0)


def _copy_out_stripe(accum, out_slice_fn, s):
    @pl.when(s < NS - 1)
    def _():
        r0 = pl.multiple_of(s * ROWS_PER_TILE, 8)
        pltpu.sync_copy(accum.at[pl.ds(r0, ROWS_PER_TILE)],
                        out_slice_fn(r0, ROWS_PER_TILE))

    @pl.when(s == NS - 1)
    def _():
        r0 = (NS - 1) * ROWS_PER_TILE
        pltpu.sync_copy(accum.at[pl.ds(r0, N - r0)], out_slice_fn(r0, N - r0))


@functools.partial(
    pl.kernel,
    out_type=jax.ShapeDtypeStruct((NC, N, F), jnp.float32),
    mesh=_MESH,
    scratch_types=[
        pltpu.VMEM_SHARED((ACC_ROWS, F), jnp.float32),
        pltpu.VMEM((JB, CHUNK), jnp.int32),
        pltpu.VMEM((JB, CHUNK), jnp.int32),
        pltpu.VMEM((CHUNK, F), jnp.float32),
        pltpu.VMEM((CHUNK, F), jnp.float32),
        pltpu.SemaphoreType.DMA,
        pltpu.SemaphoreType.DMA,
    ],
)
def _sc_agg1(xx_hbm, srca_hbm, dsta_hbm, srcb_hbm, dstb_hbm, zeros_hbm,
             out_hbm, accum, src_all, dst_all, rows0, rows1, sem0, sem1):
    c = lax.axis_index("c")
    s = lax.axis_index("s")

    @pl.when(s == 0)
    def _():
        pltpu.sync_copy(zeros_hbm, accum)

    plsc.subcore_barrier()

    nch1 = jnp.where(s == NS - 1, CH1_LAST, CH1)

    @pl.when(c == 0)
    def _():
        _gather_scatter_loop(xx_hbm.at[0], accum, s, srca_hbm, dsta_hbm,
                             src_all, dst_all, rows0, rows1, sem0, sem1, CH1)

    @pl.when(c == 1)
    def _():
        _gather_scatter_loop(xx_hbm.at[1], accum, s, srcb_hbm, dstb_hbm,
                             src_all, dst_all, rows0, rows1, sem0, sem1, nch1)

    plsc.subcore_barrier()
    _copy_out_stripe(accum, lambda r0, n: out_hbm.at[c, pl.ds(r0, n)], s)


@functools.partial(
    pl.kernel,
    out_type=jax.ShapeDtypeStruct((NC, N, F), jnp.float32),
    mesh=_MESH,
    scratch_types=[
        pltpu.VMEM_SHARED((ACC_ROWS, F), jnp.float32),
        pltpu.VMEM((JB, CHUNK), jnp.int32),
        pltpu.VMEM((JB, CHUNK), jnp.int32),
        pltpu.VMEM((CHUNK, F), jnp.float32),
        pltpu.VMEM((CHUNK, F), jnp.float32),
        pltpu.SemaphoreType.DMA,
        pltpu.SemaphoreType.DMA,
    ],
)
def _sc_agg2(ha_hbm, hb_hbm, src_hbm, dst_hbm, zeros_hbm, out_hbm,
             accum, src_all, dst_all, rows0, rows1, sem0, sem1):
    c = lax.axis_index("c")
    s = lax.axis_index("s")

    @pl.when(s == 0)
    def _():
        pltpu.sync_copy(zeros_hbm, accum)

    plsc.subcore_barrier()

    nch2 = jnp.where(s == NS - 1, CH2_LAST, CH2)

    @pl.when(c == 0)
    def _():
        _gather_scatter_loop(ha_hbm, accum, s, src_hbm, dst_hbm, src_all,
                             dst_all, rows0, rows1, sem0, sem1, nch2)

    @pl.when(c == 1)
    def _():
        _gather_scatter_loop(hb_hbm, accum, s, src_hbm, dst_hbm, src_all,
                             dst_all, rows0, rows1, sem0, sem1, nch2)

    plsc.subcore_barrier()
    _copy_out_stripe(accum, lambda r0, n: out_hbm.at[c, pl.ds(r0, n)], s)


RB = 1000
GRID = N // RB

_row_spec = pl.BlockSpec((RB, F), lambda i: (i, 0))
_w_spec = pl.BlockSpec((F, H), lambda i: (0, 0))
_b_spec = pl.BlockSpec((1, H), lambda i: (0, 0))


def _tc1_body(a0, a1, x, w_rel, w_root, b, oa, ob):
    agg = a0[...] + a1[...]
    h = (jnp.dot(agg, w_rel[...], preferred_element_type=jnp.float32)
         + jnp.dot(x[...], w_root[...], preferred_element_type=jnp.float32)
         + b[...])
    h = jnp.maximum(h, 0.0)
    oa[...] = h[:, :F]
    ob[...] = h[:, F:]


def _tc1(a0, a1, x, w_rel, w_root, b):
    return pl.pallas_call(
        _tc1_body,
        grid=(GRID,),
        in_specs=[_row_spec, _row_spec, _row_spec, _w_spec, _w_spec, _b_spec],
        out_specs=[_row_spec, _row_spec],
        out_shape=[jax.ShapeDtypeStruct((N, F), jnp.float32)] * 2,
    )(a0, a1, x, w_rel, w_root, b)


def _tc2_body(aa, ab, ha, hb, wr0, wr1, wq0, wq1, b, o):
    o[...] = (jnp.dot(aa[...], wr0[...], preferred_element_type=jnp.float32)
              + jnp.dot(ab[...], wr1[...], preferred_element_type=jnp.float32)
              + jnp.dot(ha[...], wq0[...], preferred_element_type=jnp.float32)
              + jnp.dot(hb[...], wq1[...], preferred_element_type=jnp.float32)
              + b[...])


def _tc2(aa, ab, ha, hb, wr0, wr1, wq0, wq1, b):
    return pl.pallas_call(
        _tc2_body,
        grid=(GRID,),
        in_specs=[_row_spec] * 4 + [_w_spec] * 4 + [_b_spec],
        out_specs=pl.BlockSpec((RB, H), lambda i: (i, 0)),
        out_shape=jax.ShapeDtypeStruct((N, H), jnp.float32),
    )(aa, ab, ha, hb, wr0, wr1, wq0, wq1, b)


def _pad_edges(src, dst, epad, lead):
    pad = epad - E
    srcp = jnp.concatenate([src, jnp.zeros((pad,), jnp.int32)])
    dstp = jnp.concatenate([dst, jnp.zeros((pad,), jnp.int32)])
    return (srcp.reshape(lead, -1, JB, CHUNK), dstp.reshape(lead, -1, JB, CHUNK))


def kernel(x, edge_index, W1_rel, b1_rel, W1_root, W2_rel, b2_rel, W2_root):
    src = edge_index[0].astype(jnp.int32)
    dst = edge_index[1].astype(jnp.int32)
    src1, dst1 = _pad_edges(src, dst, EPAD1, NC * NS)
    src1 = src1.reshape(NC, NS, OUTER1, JB, CHUNK)
    dst1 = dst1.reshape(NC, NS, OUTER1, JB, CHUNK)
    src2, dst2 = _pad_edges(src, dst, EPAD2, NS)
    zeros = jnp.zeros((ACC_ROWS, F), jnp.float32)

    b1 = b1_rel.reshape(1, H)
    b2 = b2_rel.reshape(1, H)

    xx = jnp.stack([x, x])
    p1 = _sc_agg1(xx, src1[0], dst1[0], src1[1], dst1[1], zeros)
    h1a, h1b = _tc1(p1[0], p1[1], x, W1_rel, W1_root, b1)

    a2 = _sc_agg2(h1a, h1b, src2, dst2, zeros)

    out = _tc2(a2[0], a2[1], h1a, h1b,
               W2_rel[:F], W2_rel[F:], W2_root[:F], W2_root[F:], b2)
    return out


# speedup vs baseline: 3.2832x; 1.0307x over previous
import functools

import jax
import jax.numpy as jnp
from jax import lax
from jax.experimental import pallas as pl
from jax.experimental.pallas import tpu as pltpu
from jax.experimental.pallas import tpu_sc as plsc

N = 10000
E = 320000
F = 128
H = 256

NC = 2
NS = 16
NW = NC * NS
CHUNK = 128
JB = 40

OUTER1 = 2
EPAD1 = NW * OUTER1 * JB * CHUNK

OUTER2 = 4
EPAD2 = NS * OUTER2 * JB * CHUNK

ACC_ROWS = N
CH1 = 80
CH1_LAST = 20
CH2 = 160
CH2_LAST = 100
ROWS_PER_TILE = 624

_MESH = plsc.VectorSubcoreMesh(core_axis_name="c", subcore_axis_name="s")


def _gather_scatter_loop(table_hbm, accum, lead, src_hbm, dst_hbm, src_all,
                         dst_all, rows0, rows1, sem0, sem1, total_chunks):
    def gather(i, buf, sem):
        return pltpu.async_copy(table_hbm.at[src_all.at[i]], buf, sem)

    def wait(i, buf, sem):
        pltpu.make_async_copy(table_hbm.at[src_all.at[i]], buf, sem).wait()

    def scatter(i, buf):
        pltpu.sync_copy(buf, accum.at[dst_all.at[i]], add=True)

    total_chunks = jnp.int32(total_chunks)
    nblocks = (total_chunks + (JB - 1)) // JB

    def outer_body(ob, carry):
        pltpu.sync_copy(src_hbm.at[lead, ob], src_all)
        pltpu.sync_copy(dst_hbm.at[lead, ob], dst_all)
        npair = jnp.minimum(JB, total_chunks - ob * JB) // 2
        gather(0, rows0, sem0)

        def step(j, c2):
            i0 = j * 2
            gather(i0 + 1, rows1, sem1)
            wait(i0, rows0, sem0)
            scatter(i0, rows0)

            @pl.when(j + 1 < npair)
            def _():
                gather(i0 + 2, rows0, sem0)

            wait(i0 + 1, rows1, sem1)
            scatter(i0 + 1, rows1)
            return c2

        lax.fori_loop(0, npair, step, 0)
        return carry

    lax.fori_loop(0, nblocks, outer_body, 0)


def _copy_out_stripe(accum, out_slice_fn, s):
    @pl.when(s < NS - 1)
    def _():
        r0 = pl.multiple_of(s * ROWS_PER_TILE, 8)
        pltpu.sync_copy(accum.at[pl.ds(r0, ROWS_PER_TILE)],
                        out_slice_fn(r0, ROWS_PER_TILE))

    @pl.when(s == NS - 1)
    def _():
        r0 = (NS - 1) * ROWS_PER_TILE
        pltpu.sync_copy(accum.at[pl.ds(r0, N - r0)], out_slice_fn(r0, N - r0))


@functools.partial(
    pl.kernel,
    out_type=[jax.ShapeDtypeStruct((N, F), jnp.float32)] * 2,
    mesh=_MESH,
    scratch_types=[
        pltpu.VMEM_SHARED((ACC_ROWS, F), jnp.float32),
        pltpu.VMEM((JB, CHUNK), jnp.int32),
        pltpu.VMEM((JB, CHUNK), jnp.int32),
        pltpu.VMEM((CHUNK, F), jnp.float32),
        pltpu.VMEM((CHUNK, F), jnp.float32),
        pltpu.SemaphoreType.DMA,
        pltpu.SemaphoreType.DMA,
    ],
)
def _sc_agg1(xx_hbm, srca_hbm, dsta_hbm, srcb_hbm, dstb_hbm, zeros_hbm,
             outa_hbm, outb_hbm, accum, src_all, dst_all, rows0, rows1,
             sem0, sem1):
    c = lax.axis_index("c")
    s = lax.axis_index("s")

    @pl.when(s == 0)
    def _():
        pltpu.sync_copy(zeros_hbm, accum)

    plsc.subcore_barrier()

    nch1 = jnp.where(s == NS - 1, CH1_LAST, CH1)

    @pl.when(c == 0)
    def _():
        _gather_scatter_loop(xx_hbm.at[0], accum, s, srca_hbm, dsta_hbm,
                             src_all, dst_all, rows0, rows1, sem0, sem1, CH1)

    @pl.when(c == 1)
    def _():
        _gather_scatter_loop(xx_hbm.at[1], accum, s, srcb_hbm, dstb_hbm,
                             src_all, dst_all, rows0, rows1, sem0, sem1, nch1)

    plsc.subcore_barrier()

    @pl.when(c == 0)
    def _():
        _copy_out_stripe(accum, lambda r0, n: outa_hbm.at[pl.ds(r0, n)], s)

    @pl.when(c == 1)
    def _():
        _copy_out_stripe(accum, lambda r0, n: outb_hbm.at[pl.ds(r0, n)], s)


@functools.partial(
    pl.kernel,
    out_type=[jax.ShapeDtypeStruct((N, F), jnp.float32)] * 2,
    mesh=_MESH,
    scratch_types=[
        pltpu.VMEM_SHARED((ACC_ROWS, F), jnp.float32),
        pltpu.VMEM((JB, CHUNK), jnp.int32),
        pltpu.VMEM((JB, CHUNK), jnp.int32),
        pltpu.VMEM((CHUNK, F), jnp.float32),
        pltpu.VMEM((CHUNK, F), jnp.float32),
        pltpu.SemaphoreType.DMA,
        pltpu.SemaphoreType.DMA,
    ],
)
def _sc_agg2(ha_hbm, hb_hbm, src_hbm, dst_hbm, zeros_hbm,
             outa_hbm, outb_hbm, accum, src_all, dst_all, rows0, rows1,
             sem0, sem1):
    c = lax.axis_index("c")
    s = lax.axis_index("s")

    @pl.when(s == 0)
    def _():
        pltpu.sync_copy(zeros_hbm, accum)

    plsc.subcore_barrier()

    nch2 = jnp.where(s == NS - 1, CH2_LAST, CH2)

    @pl.when(c == 0)
    def _():
        _gather_scatter_loop(ha_hbm, accum, s, src_hbm, dst_hbm, src_all,
                             dst_all, rows0, rows1, sem0, sem1, nch2)

    @pl.when(c == 1)
    def _():
        _gather_scatter_loop(hb_hbm, accum, s, src_hbm, dst_hbm, src_all,
                             dst_all, rows0, rows1, sem0, sem1, nch2)

    plsc.subcore_barrier()

    @pl.when(c == 0)
    def _():
        _copy_out_stripe(accum, lambda r0, n: outa_hbm.at[pl.ds(r0, n)], s)

    @pl.when(c == 1)
    def _():
        _copy_out_stripe(accum, lambda r0, n: outb_hbm.at[pl.ds(r0, n)], s)


RB = 1000
GRID = N // RB

_row_spec = pl.BlockSpec((RB, F), lambda i: (i, 0))
_w_spec = pl.BlockSpec((F, H), lambda i: (0, 0))
_b_spec = pl.BlockSpec((1, H), lambda i: (0, 0))


def _tc_root1_body(x, w_root, b, o):
    o[...] = (jnp.dot(x[...], w_root[...], preferred_element_type=jnp.float32)
              + b[...])


def _tc_root1(x, w_root, b):
    return pl.pallas_call(
        _tc_root1_body,
        grid=(GRID,),
        in_specs=[_row_spec, _w_spec, _b_spec],
        out_specs=pl.BlockSpec((RB, H), lambda i: (i, 0)),
        out_shape=jax.ShapeDtypeStruct((N, H), jnp.float32),
    )(x, w_root, b)


def _tc_fin1_body(a0, a1, xr, w_rel, oa, ob):
    agg = a0[...] + a1[...]
    h = jnp.dot(agg, w_rel[...], preferred_element_type=jnp.float32) + xr[...]
    h = jnp.maximum(h, 0.0)
    oa[...] = h[:, :F]
    ob[...] = h[:, F:]


def _tc_fin1(a0, a1, xr, w_rel):
    return pl.pallas_call(
        _tc_fin1_body,
        grid=(GRID,),
        in_specs=[_row_spec, _row_spec,
                  pl.BlockSpec((RB, H), lambda i: (i, 0)), _w_spec],
        out_specs=[_row_spec, _row_spec],
        out_shape=[jax.ShapeDtypeStruct((N, F), jnp.float32)] * 2,
    )(a0, a1, xr, w_rel)


def _tc_root2_body(ha, hb, wq0, wq1, b, o):
    o[...] = (jnp.dot(ha[...], wq0[...], preferred_element_type=jnp.float32)
              + jnp.dot(hb[...], wq1[...], preferred_element_type=jnp.float32)
              + b[...])


def _tc_root2(ha, hb, wq0, wq1, b):
    return pl.pallas_call(
        _tc_root2_body,
        grid=(GRID,),
        in_specs=[_row_spec, _row_spec, _w_spec, _w_spec, _b_spec],
        out_specs=pl.BlockSpec((RB, H), lambda i: (i, 0)),
        out_shape=jax.ShapeDtypeStruct((N, H), jnp.float32),
    )(ha, hb, wq0, wq1, b)


def _tc_fin2_body(aa, ab, hr, wr0, wr1, o):
    o[...] = (jnp.dot(aa[...], wr0[...], preferred_element_type=jnp.float32)
              + jnp.dot(ab[...], wr1[...], preferred_element_type=jnp.float32)
              + hr[...])


def _tc_fin2(aa, ab, hr, wr0, wr1):
    return pl.pallas_call(
        _tc_fin2_body,
        grid=(GRID,),
        in_specs=[_row_spec, _row_spec,
                  pl.BlockSpec((RB, H), lambda i: (i, 0)), _w_spec, _w_spec],
        out_specs=pl.BlockSpec((RB, H), lambda i: (i, 0)),
        out_shape=jax.ShapeDtypeStruct((N, H), jnp.float32),
    )(aa, ab, hr, wr0, wr1)


def _pad_edges(src, dst, epad, lead):
    pad = epad - E
    srcp = jnp.concatenate([src, jnp.zeros((pad,), jnp.int32)])
    dstp = jnp.concatenate([dst, jnp.zeros((pad,), jnp.int32)])
    return (srcp.reshape(lead, -1, JB, CHUNK), dstp.reshape(lead, -1, JB, CHUNK))


def kernel(x, edge_index, W1_rel, b1_rel, W1_root, W2_rel, b2_rel, W2_root):
    src = edge_index[0].astype(jnp.int32)
    dst = edge_index[1].astype(jnp.int32)
    src1, dst1 = _pad_edges(src, dst, EPAD1, NC * NS)
    src1 = src1.reshape(NC, NS, OUTER1, JB, CHUNK)
    dst1 = dst1.reshape(NC, NS, OUTER1, JB, CHUNK)
    src2, dst2 = _pad_edges(src, dst, EPAD2, NS)
    zeros = jnp.zeros((ACC_ROWS, F), jnp.float32)

    b1 = b1_rel.reshape(1, H)
    b2 = b2_rel.reshape(1, H)

    xx = jnp.stack([x, x])
    p0, p1 = _sc_agg1(xx, src1[0], dst1[0], src1[1], dst1[1], zeros)
    xr = _tc_root1(x, W1_root, b1)
    h1a, h1b = _tc_fin1(p0, p1, xr, W1_rel)

    a0, a1 = _sc_agg2(h1a, h1b, src2, dst2, zeros)
    hr = _tc_root2(h1a, h1b, W2_root[:F], W2_root[F:], b2)

    out = _tc_fin2(a0, a1, hr, W2_rel[:F], W2_rel[F:])
    return out


# speedup vs baseline: 3.4264x; 1.0436x over previous
import functools

import jax
import jax.numpy as jnp
from jax import lax
from jax.experimental import pallas as pl
from jax.experimental.pallas import tpu as pltpu
from jax.experimental.pallas import tpu_sc as plsc

N = 10000
E = 320000
F = 128
H = 256

NC = 2
NS = 16
CHUNK = 128
NCHUNKS = E // CHUNK
JB = 40

C1_PER_CORE = NCHUNKS // NC

ACC_ROWS = N
ROWS_PER_TILE = 624

_MESH = plsc.VectorSubcoreMesh(core_axis_name="c", subcore_axis_name="s")


def _gather_scatter_loop(table_hbm, accum, e_hbm, base, total,
                         src_all, dst_all, rows0, rows1, sem0, sem1):
    base = jnp.int32(base)
    total = jnp.int32(total)
    nblocks = (total + (JB - 1)) // JB

    def gather(row, buf, sem):
        pltpu.async_copy(table_hbm.at[src_all.at[row, 0]], buf, sem)

    def wait_g(row, buf, sem):
        pltpu.make_async_copy(table_hbm.at[src_all.at[row, 0]], buf, sem).wait()

    def scatter(row, buf):
        pltpu.sync_copy(buf, accum.at[dst_all.at[row, 0]], add=True)

    def outer_body(ob, carry):
        bstart = base + ob * JB
        load_base = jnp.minimum(bstart, NCHUNKS - JB)
        shift = bstart - load_base
        cnt = jnp.minimum(JB, total - ob * JB)
        npair = cnt // 2
        pltpu.sync_copy(e_hbm.at[0, pl.ds(load_base, JB)], src_all)
        pltpu.sync_copy(e_hbm.at[1, pl.ds(load_base, JB)], dst_all)
        gather(shift, rows0, sem0)

        def step(j, c2):
            r0 = shift + j * 2
            gather(r0 + 1, rows1, sem1)
            wait_g(r0, rows0, sem0)
            scatter(r0, rows0)

            @pl.when(j + 1 < npair)
            def _():
                gather(r0 + 2, rows0, sem0)

            wait_g(r0 + 1, rows1, sem1)
            scatter(r0 + 1, rows1)
            return c2

        lax.fori_loop(0, npair, step, 0)

        @pl.when(cnt % 2 == 1)
        def _():
            r = shift + cnt - 1
            gather(r, rows0, sem0)
            wait_g(r, rows0, sem0)
            scatter(r, rows0)

        return carry

    lax.fori_loop(0, nblocks, outer_body, 0)


def _copy_out_stripe(accum, out_slice_fn, s):
    @pl.when(s < NS - 1)
    def _():
        r0 = pl.multiple_of(s * ROWS_PER_TILE, 8)
        pltpu.sync_copy(accum.at[pl.ds(r0, ROWS_PER_TILE)],
                        out_slice_fn(r0, ROWS_PER_TILE))

    @pl.when(s == NS - 1)
    def _():
        r0 = (NS - 1) * ROWS_PER_TILE
        pltpu.sync_copy(accum.at[pl.ds(r0, N - r0)], out_slice_fn(r0, N - r0))


_SC_SCRATCH = [
    pltpu.VMEM_SHARED((ACC_ROWS, F), jnp.float32),
    pltpu.VMEM((JB, 1, CHUNK), jnp.int32),
    pltpu.VMEM((JB, 1, CHUNK), jnp.int32),
    pltpu.VMEM((CHUNK, F), jnp.float32),
    pltpu.VMEM((CHUNK, F), jnp.float32),
    pltpu.SemaphoreType.DMA,
    pltpu.SemaphoreType.DMA,
]


@functools.partial(
    pl.kernel,
    out_type=[jax.ShapeDtypeStruct((N, F), jnp.float32)] * 2,
    mesh=_MESH,
    scratch_types=_SC_SCRATCH,
)
def _sc_agg1(x_hbm, e_hbm, zeros_hbm, outa_hbm, outb_hbm,
             accum, src_all, dst_all, rows0, rows1, sem0, sem1):
    c = lax.axis_index("c")
    s = lax.axis_index("s")

    @pl.when(s == 0)
    def _():
        pltpu.sync_copy(zeros_hbm, accum)

    plsc.subcore_barrier()

    base = c * C1_PER_CORE + s * 78 + jnp.minimum(s, 2)
    total = jnp.where(s < 2, 79, 78)
    _gather_scatter_loop(x_hbm, accum, e_hbm, base, total,
                         src_all, dst_all, rows0, rows1, sem0, sem1)

    plsc.subcore_barrier()

    @pl.when(c == 0)
    def _():
        _copy_out_stripe(accum, lambda r0, n: outa_hbm.at[pl.ds(r0, n)], s)

    @pl.when(c == 1)
    def _():
        _copy_out_stripe(accum, lambda r0, n: outb_hbm.at[pl.ds(r0, n)], s)


@functools.partial(
    pl.kernel,
    out_type=[jax.ShapeDtypeStruct((N, F), jnp.float32)] * 2,
    mesh=_MESH,
    scratch_types=_SC_SCRATCH,
)
def _sc_agg2(ha_hbm, hb_hbm, e_hbm, zeros_hbm, outa_hbm, outb_hbm,
             accum, src_all, dst_all, rows0, rows1, sem0, sem1):
    c = lax.axis_index("c")
    s = lax.axis_index("s")

    @pl.when(s == 0)
    def _():
        pltpu.sync_copy(zeros_hbm, accum)

    plsc.subcore_barrier()

    base = s * 156 + jnp.minimum(s, 4)
    total = jnp.where(s < 4, 157, 156)

    @pl.when(c == 0)
    def _():
        _gather_scatter_loop(ha_hbm, accum, e_hbm, base, total,
                             src_all, dst_all, rows0, rows1, sem0, sem1)

    @pl.when(c == 1)
    def _():
        _gather_scatter_loop(hb_hbm, accum, e_hbm, base, total,
                             src_all, dst_all, rows0, rows1, sem0, sem1)

    plsc.subcore_barrier()

    @pl.when(c == 0)
    def _():
        _copy_out_stripe(accum, lambda r0, n: outa_hbm.at[pl.ds(r0, n)], s)

    @pl.when(c == 1)
    def _():
        _copy_out_stripe(accum, lambda r0, n: outb_hbm.at[pl.ds(r0, n)], s)


RB = 1000
GRID = N // RB

_row_spec = pl.BlockSpec((RB, F), lambda i: (i, 0))
_wide_spec = pl.BlockSpec((RB, H), lambda i: (i, 0))
_w_spec = pl.BlockSpec((F, H), lambda i: (0, 0))
_b_spec = pl.BlockSpec((1, H), lambda i: (0, 0))


def _tc_root1_body(x, w_root, b, o):
    o[...] = (jnp.dot(x[...], w_root[...], preferred_element_type=jnp.float32)
              + b[...])


def _tc_root1(x, w_root, b):
    return pl.pallas_call(
        _tc_root1_body,
        grid=(GRID,),
        in_specs=[_row_spec, _w_spec, _b_spec],
        out_specs=_wide_spec,
        out_shape=jax.ShapeDtypeStruct((N, H), jnp.float32),
    )(x, w_root, b)


def _tc_fin1_body(a0, a1, xr, w_rel, oa, ob):
    agg = a0[...] + a1[...]
    h = jnp.dot(agg, w_rel[...], preferred_element_type=jnp.float32) + xr[...]
    h = jnp.maximum(h, 0.0)
    oa[...] = h[:, :F]
    ob[...] = h[:, F:]


def _tc_fin1(a0, a1, xr, w_rel):
    return pl.pallas_call(
        _tc_fin1_body,
        grid=(GRID,),
        in_specs=[_row_spec, _row_spec, _wide_spec, _w_spec],
        out_specs=[_row_spec, _row_spec],
        out_shape=[jax.ShapeDtypeStruct((N, F), jnp.float32)] * 2,
    )(a0, a1, xr, w_rel)


def _tc_root2_body(ha, hb, wq0, wq1, b, o):
    o[...] = (jnp.dot(ha[...], wq0[...], preferred_element_type=jnp.float32)
              + jnp.dot(hb[...], wq1[...], preferred_element_type=jnp.float32)
              + b[...])


def _tc_root2(ha, hb, wq0, wq1, b):
    return pl.pallas_call(
        _tc_root2_body,
        grid=(GRID,),
        in_specs=[_row_spec, _row_spec, _w_spec, _w_spec, _b_spec],
        out_specs=_wide_spec,
        out_shape=jax.ShapeDtypeStruct((N, H), jnp.float32),
    )(ha, hb, wq0, wq1, b)


def _tc_fin2_body(aa, ab, hr, wr0, wr1, o):
    o[...] = (jnp.dot(aa[...], wr0[...], preferred_element_type=jnp.float32)
              + jnp.dot(ab[...], wr1[...], preferred_element_type=jnp.float32)
              + hr[...])


def _tc_fin2(aa, ab, hr, wr0, wr1):
    return pl.pallas_call(
        _tc_fin2_body,
        grid=(GRID,),
        in_specs=[_row_spec, _row_spec, _wide_spec, _w_spec, _w_spec],
        out_specs=_wide_spec,
        out_shape=jax.ShapeDtypeStruct((N, H), jnp.float32),
    )(aa, ab, hr, wr0, wr1)


def kernel(x, edge_index, W1_rel, b1_rel, W1_root, W2_rel, b2_rel, W2_root):
    e4 = edge_index.astype(jnp.int32).reshape(2, NCHUNKS, 1, CHUNK)
    zeros = jnp.zeros((ACC_ROWS, F), jnp.float32)
    b1 = b1_rel.reshape(1, H)
    b2 = b2_rel.reshape(1, H)

    p0, p1 = _sc_agg1(x, e4, zeros)
    xr = _tc_root1(x, W1_root, b1)
    h1a, h1b = _tc_fin1(p0, p1, xr, W1_rel)

    a0, a1 = _sc_agg2(h1a, h1b, e4, zeros)
    hr = _tc_root2(h1a, h1b, W2_root[:F], W2_root[F:], b2)

    out = _tc_fin2(a0, a1, hr, W2_rel[:F], W2_rel[F:])
    return out


# speedup vs baseline: 3.4515x; 1.0073x over previous
import functools

import jax
import jax.numpy as jnp
from jax import lax
from jax.experimental import pallas as pl
from jax.experimental.pallas import tpu as pltpu
from jax.experimental.pallas import tpu_sc as plsc

N = 10000
E = 320000
F = 128
H = 256

NC = 2
NS = 16
CHUNK = 128
NCHUNKS = E // CHUNK
PCHUNKS = 2560
JB = 40

ACC_ROWS = N
ROWS_PER_TILE = 624

_MESH = plsc.VectorSubcoreMesh(core_axis_name="c", subcore_axis_name="s")


def _gather_scatter_loop(table_hbm, accum, e_hbm, base, total,
                         src_all, dst_all, rows0, rows1, sem0, sem1):
    base = jnp.int32(base)
    total = jnp.int32(total)
    nblocks = (total + (JB - 1)) // JB

    def gather(row, buf, sem):
        pltpu.async_copy(table_hbm.at[src_all.at[row]], buf, sem)

    def wait_g(row, buf, sem):
        pltpu.make_async_copy(table_hbm.at[src_all.at[row]], buf, sem).wait()

    def scatter(row, buf):
        pltpu.sync_copy(buf, accum.at[dst_all.at[row]], add=True)

    def outer_body(ob, carry):
        bstart = pl.multiple_of(base + ob * JB, 8)
        npair = jnp.minimum(JB, total - ob * JB) // 2
        pltpu.sync_copy(e_hbm.at[0, pl.ds(bstart, JB)], src_all)
        pltpu.sync_copy(e_hbm.at[1, pl.ds(bstart, JB)], dst_all)
        gather(0, rows0, sem0)

        def step(j, c2):
            r0 = j * 2
            gather(r0 + 1, rows1, sem1)
            wait_g(r0, rows0, sem0)
            scatter(r0, rows0)

            @pl.when(j + 1 < npair)
            def _():
                gather(r0 + 2, rows0, sem0)

            wait_g(r0 + 1, rows1, sem1)
            scatter(r0 + 1, rows1)
            return c2

        lax.fori_loop(0, npair, step, 0)
        return carry

    lax.fori_loop(0, nblocks, outer_body, 0)


def _copy_out_stripe(accum, out_slice_fn, s):
    @pl.when(s < NS - 1)
    def _():
        r0 = pl.multiple_of(s * ROWS_PER_TILE, 8)
        pltpu.sync_copy(accum.at[pl.ds(r0, ROWS_PER_TILE)],
                        out_slice_fn(r0, ROWS_PER_TILE))

    @pl.when(s == NS - 1)
    def _():
        r0 = (NS - 1) * ROWS_PER_TILE
        pltpu.sync_copy(accum.at[pl.ds(r0, N - r0)], out_slice_fn(r0, N - r0))


_SC_SCRATCH = [
    pltpu.VMEM_SHARED((ACC_ROWS, F), jnp.float32),
    pltpu.VMEM((JB, CHUNK), jnp.int32),
    pltpu.VMEM((JB, CHUNK), jnp.int32),
    pltpu.VMEM((CHUNK, F), jnp.float32),
    pltpu.VMEM((CHUNK, F), jnp.float32),
    pltpu.SemaphoreType.DMA,
    pltpu.SemaphoreType.DMA,
]


@functools.partial(
    pl.kernel,
    out_type=[jax.ShapeDtypeStruct((N, F), jnp.float32)] * 2,
    mesh=_MESH,
    scratch_types=_SC_SCRATCH,
)
def _sc_agg1(x_hbm, e_hbm, zeros_hbm, outa_hbm, outb_hbm,
             accum, src_all, dst_all, rows0, rows1, sem0, sem1):
    c = lax.axis_index("c")
    s = lax.axis_index("s")

    @pl.when(s == 0)
    def _():
        pltpu.sync_copy(zeros_hbm, accum)

    plsc.subcore_barrier()

    base = c * (PCHUNKS // 2) + s * 80
    total = jnp.where((c == 1) & (s == NS - 1), 20, 80)
    _gather_scatter_loop(x_hbm, accum, e_hbm, base, total,
                         src_all, dst_all, rows0, rows1, sem0, sem1)

    plsc.subcore_barrier()

    @pl.when(c == 0)
    def _():
        _copy_out_stripe(accum, lambda r0, n: outa_hbm.at[pl.ds(r0, n)], s)

    @pl.when(c == 1)
    def _():
        _copy_out_stripe(accum, lambda r0, n: outb_hbm.at[pl.ds(r0, n)], s)


@functools.partial(
    pl.kernel,
    out_type=[jax.ShapeDtypeStruct((N, F), jnp.float32)] * 2,
    mesh=_MESH,
    scratch_types=_SC_SCRATCH,
)
def _sc_agg2(ha_hbm, hb_hbm, e_hbm, zeros_hbm, outa_hbm, outb_hbm,
             accum, src_all, dst_all, rows0, rows1, sem0, sem1):
    c = lax.axis_index("c")
    s = lax.axis_index("s")

    @pl.when(s == 0)
    def _():
        pltpu.sync_copy(zeros_hbm, accum)

    plsc.subcore_barrier()

    base = s * (PCHUNKS // NS)
    total = jnp.where(s == NS - 1, 100, 160)

    @pl.when(c == 0)
    def _():
        _gather_scatter_loop(ha_hbm, accum, e_hbm, base, total,
                             src_all, dst_all, rows0, rows1, sem0, sem1)

    @pl.when(c == 1)
    def _():
        _gather_scatter_loop(hb_hbm, accum, e_hbm, base, total,
                             src_all, dst_all, rows0, rows1, sem0, sem1)

    plsc.subcore_barrier()

    @pl.when(c == 0)
    def _():
        _copy_out_stripe(accum, lambda r0, n: outa_hbm.at[pl.ds(r0, n)], s)

    @pl.when(c == 1)
    def _():
        _copy_out_stripe(accum, lambda r0, n: outb_hbm.at[pl.ds(r0, n)], s)


RB = 1000
GRID = N // RB

_row_spec = pl.BlockSpec((RB, F), lambda i: (i, 0))
_wide_spec = pl.BlockSpec((RB, H), lambda i: (i, 0))
_w_spec = pl.BlockSpec((F, H), lambda i: (0, 0))
_b_spec = pl.BlockSpec((1, H), lambda i: (0, 0))


def _tc_root1_body(x, w_root, b, o):
    o[...] = (jnp.dot(x[...], w_root[...], preferred_element_type=jnp.float32)
              + b[...])


def _tc_root1(x, w_root, b):
    return pl.pallas_call(
        _tc_root1_body,
        grid=(GRID,),
        in_specs=[_row_spec, _w_spec, _b_spec],
        out_specs=_wide_spec,
        out_shape=jax.ShapeDtypeStruct((N, H), jnp.float32),
    )(x, w_root, b)


def _tc_fin1_body(a0, a1, xr, w_rel, oa, ob):
    agg = a0[...] + a1[...]
    h = jnp.dot(agg, w_rel[...], preferred_element_type=jnp.float32) + xr[...]
    h = jnp.maximum(h, 0.0)
    oa[...] = h[:, :F]
    ob[...] = h[:, F:]


def _tc_fin1(a0, a1, xr, w_rel):
    return pl.pallas_call(
        _tc_fin1_body,
        grid=(GRID,),
        in_specs=[_row_spec, _row_spec, _wide_spec, _w_spec],
        out_specs=[_row_spec, _row_spec],
        out_shape=[jax.ShapeDtypeStruct((N, F), jnp.float32)] * 2,
    )(a0, a1, xr, w_rel)


def _tc_root2_body(ha, hb, wq0, wq1, b, o):
    o[...] = (jnp.dot(ha[...], wq0[...], preferred_element_type=jnp.float32)
              + jnp.dot(hb[...], wq1[...], preferred_element_type=jnp.float32)
              + b[...])


def _tc_root2(ha, hb, wq0, wq1, b):
    return pl.pallas_call(
        _tc_root2_body,
        grid=(GRID,),
        in_specs=[_row_spec, _row_spec, _w_spec, _w_spec, _b_spec],
        out_specs=_wide_spec,
        out_shape=jax.ShapeDtypeStruct((N, H), jnp.float32),
    )(ha, hb, wq0, wq1, b)


def _tc_fin2_body(aa, ab, hr, wr0, wr1, o):
    o[...] = (jnp.dot(aa[...], wr0[...], preferred_element_type=jnp.float32)
              + jnp.dot(ab[...], wr1[...], preferred_element_type=jnp.float32)
              + hr[...])


def _tc_fin2(aa, ab, hr, wr0, wr1):
    return pl.pallas_call(
        _tc_fin2_body,
        grid=(GRID,),
        in_specs=[_row_spec, _row_spec, _wide_spec, _w_spec, _w_spec],
        out_specs=_wide_spec,
        out_shape=jax.ShapeDtypeStruct((N, H), jnp.float32),
    )(aa, ab, hr, wr0, wr1)


def kernel(x, edge_index, W1_rel, b1_rel, W1_root, W2_rel, b2_rel, W2_root):
    ep = jnp.pad(edge_index.astype(jnp.int32),
                 ((0, 0), (0, (PCHUNKS - NCHUNKS) * CHUNK)))
    e4 = ep.reshape(2, PCHUNKS, CHUNK)
    zeros = jnp.zeros((ACC_ROWS, F), jnp.float32)
    b1 = b1_rel.reshape(1, H)
    b2 = b2_rel.reshape(1, H)

    p0, p1 = _sc_agg1(x, e4, zeros)
    xr = _tc_root1(x, W1_root, b1)
    h1a, h1b = _tc_fin1(p0, p1, xr, W1_rel)

    a0, a1 = _sc_agg2(h1a, h1b, e4, zeros)
    hr = _tc_root2(h1a, h1b, W2_root[:F], W2_root[F:], b2)

    out = _tc_fin2(a0, a1, hr, W2_rel[:F], W2_rel[F:])
    return out
